# baseline (device time: 1455000 ns/iter reference)
import jax
import jax.numpy as jnp
from jax import lax
from jax.experimental import pallas as pl
from jax.experimental.pallas import tpu as pltpu

N_DEV = 16
S = 4096
D = 1024
CH = S // N_DEV


def _ring_allreduce(partial):

    def body(x_ref, out_ref, comm, send_sems, recv_sems, credit_sem):
        my = lax.axis_index("i")
        left = jnp.mod(my - 1, N_DEV)
        right = jnp.mod(my + 1, N_DEV)

        out_ref[...] = x_ref[...]

        n_steps = 2 * (N_DEV - 1)
        for step in range(n_steps):
            slot = step % 2
            is_rs = step < N_DEV - 1
            if is_rs:
                send_c = jnp.mod(my - step, N_DEV)
                recv_c = jnp.mod(my - step - 1, N_DEV)
                dst = comm.at[slot]
            else:
                t = step - (N_DEV - 1)
                send_c = jnp.mod(my + 1 - t, N_DEV)
                recv_c = jnp.mod(my - t, N_DEV)
                dst = out_ref.at[pl.ds(send_c * CH, CH), :]

            if step >= 2:
                pl.semaphore_wait(credit_sem, 1)

            rdma = pltpu.make_async_remote_copy(
                src_ref=out_ref.at[pl.ds(send_c * CH, CH), :],
                dst_ref=dst,
                send_sem=send_sems.at[slot],
                recv_sem=recv_sems.at[slot],
                device_id=(right,),
                device_id_type=pl.DeviceIdType.MESH,
            )
            rdma.start()
            rdma.wait()

            if is_rs:
                out_ref[pl.ds(recv_c * CH, CH), :] += comm[slot]

            pl.semaphore_signal(
                credit_sem,
                inc=1,
                device_id=(left,),
                device_id_type=pl.DeviceIdType.MESH,
            )

        pl.semaphore_wait(credit_sem, 2)

    return pl.pallas_call(
        body,
        out_shape=jax.ShapeDtypeStruct((S, D), jnp.float32),
        in_specs=[pl.BlockSpec(memory_space=pltpu.VMEM)],
        out_specs=pl.BlockSpec(memory_space=pltpu.VMEM),
        scratch_shapes=[
            pltpu.VMEM((2, CH, D), jnp.float32),
            pltpu.SemaphoreType.DMA((2,)),
            pltpu.SemaphoreType.DMA((2,)),
            pltpu.SemaphoreType.REGULAR,
        ],
    )(partial)


def kernel(x, Wq, Wk, Wv, Wo, t_emb, W_mod, W_ff1, W_ff2):
    eps = 1e-5
    mod = t_emb @ W_mod
    sa, sha, ga, sm, shm, gm = jnp.split(mod, 6, axis=-1)

    def ln(h):
        m = h.mean(axis=-1, keepdims=True)
        v = h.var(axis=-1, keepdims=True)
        return (h - m) * lax.rsqrt(v + eps)

    x0 = x[0]
    xln = ln(x0) * (1.0 + sa) + sha

    n_local_heads = Wq.shape[1] // 128
    Q = (xln @ Wq).reshape(S, n_local_heads, 128)
    K = (xln @ Wk).reshape(S, n_local_heads, 128)
    V = (xln @ Wv).reshape(S, n_local_heads, 128)

    scores = jnp.einsum("ihd,jhd->hij", Q, K) * 0.08838834764831843
    p = jax.nn.softmax(scores, axis=-1)
    attn = jnp.einsum("hij,jhd->ihd", p, V).reshape(S, n_local_heads * 128)

    attn_out = _ring_allreduce(attn @ Wo)
    x_mid = x0 + ga * attn_out

    xln2 = ln(x_mid) * (1.0 + sm) + shm
    h = xln2 @ W_ff1
    h = h * jax.nn.sigmoid(h)
    ffn = _ring_allreduce(h @ W_ff2)
    out = x_mid + gm * ffn
    return out[None]


# device time: 1193577 ns/iter; 1.2190x vs baseline; 1.2190x over previous
import jax
import jax.numpy as jnp
from jax import lax
from jax.experimental import pallas as pl
from jax.experimental.pallas import tpu as pltpu

N_DEV = 16
S = 4096
D = 1024
HC = S // (2 * N_DEV)
H1 = S // 2


def _ring_allreduce(partial):

    def body(
        x_ref,
        out_ref,
        comm_cw,
        comm_ccw,
        send_cw,
        recv_cw,
        send_ccw,
        recv_ccw,
        credit_cw,
        credit_ccw,
    ):
        my = lax.axis_index("i")
        left = jnp.mod(my - 1, N_DEV)
        right = jnp.mod(my + 1, N_DEV)

        out_ref[...] = x_ref[...]

        n_steps = 2 * (N_DEV - 1)
        for step in range(n_steps):
            slot = step % 2
            is_rs = step < N_DEV - 1
            if is_rs:
                s_cw = jnp.mod(my - step, N_DEV)
                r_cw = jnp.mod(my - step - 1, N_DEV)
                s_ccw = jnp.mod(my + step, N_DEV)
                r_ccw = jnp.mod(my + step + 1, N_DEV)
                dst_cw = comm_cw.at[slot]
                dst_ccw = comm_ccw.at[slot]
            else:
                t = step - (N_DEV - 1)
                s_cw = jnp.mod(my + 1 - t, N_DEV)
                r_cw = jnp.mod(my - t, N_DEV)
                s_ccw = jnp.mod(my - 1 + t, N_DEV)
                r_ccw = jnp.mod(my + t, N_DEV)
                dst_cw = out_ref.at[pl.ds(s_cw * HC, HC), :]
                dst_ccw = out_ref.at[pl.ds(H1 + s_ccw * HC, HC), :]

            if step >= 2:
                pl.semaphore_wait(credit_cw, 1)
                pl.semaphore_wait(credit_ccw, 1)

            rdma_cw = pltpu.make_async_remote_copy(
                src_ref=out_ref.at[pl.ds(s_cw * HC, HC), :],
                dst_ref=dst_cw,
                send_sem=send_cw.at[slot],
                recv_sem=recv_cw.at[slot],
                device_id=(right,),
                device_id_type=pl.DeviceIdType.MESH,
            )
            rdma_ccw = pltpu.make_async_remote_copy(
                src_ref=out_ref.at[pl.ds(H1 + s_ccw * HC, HC), :],
                dst_ref=dst_ccw,
                send_sem=send_ccw.at[slot],
                recv_sem=recv_ccw.at[slot],
                device_id=(left,),
                device_id_type=pl.DeviceIdType.MESH,
            )
            rdma_cw.start()
            rdma_ccw.start()
            rdma_cw.wait()
            rdma_ccw.wait()

            if is_rs:
                out_ref[pl.ds(r_cw * HC, HC), :] += comm_cw[slot]
                out_ref[pl.ds(H1 + r_ccw * HC, HC), :] += comm_ccw[slot]

            pl.semaphore_signal(
                credit_cw,
                inc=1,
                device_id=(left,),
                device_id_type=pl.DeviceIdType.MESH,
            )
            pl.semaphore_signal(
                credit_ccw,
                inc=1,
                device_id=(right,),
                device_id_type=pl.DeviceIdType.MESH,
            )

        pl.semaphore_wait(credit_cw, 2)
        pl.semaphore_wait(credit_ccw, 2)

    return pl.pallas_call(
        body,
        out_shape=jax.ShapeDtypeStruct((S, D), jnp.float32),
        in_specs=[pl.BlockSpec(memory_space=pltpu.VMEM)],
        out_specs=pl.BlockSpec(memory_space=pltpu.VMEM),
        scratch_shapes=[
            pltpu.VMEM((2, HC, D), jnp.float32),
            pltpu.VMEM((2, HC, D), jnp.float32),
            pltpu.SemaphoreType.DMA((2,)),
            pltpu.SemaphoreType.DMA((2,)),
            pltpu.SemaphoreType.DMA((2,)),
            pltpu.SemaphoreType.DMA((2,)),
            pltpu.SemaphoreType.REGULAR,
            pltpu.SemaphoreType.REGULAR,
        ],
    )(partial)


def kernel(x, Wq, Wk, Wv, Wo, t_emb, W_mod, W_ff1, W_ff2):
    eps = 1e-5
    mod = t_emb @ W_mod
    sa, sha, ga, sm, shm, gm = jnp.split(mod, 6, axis=-1)

    def ln(h):
        m = h.mean(axis=-1, keepdims=True)
        v = h.var(axis=-1, keepdims=True)
        return (h - m) * lax.rsqrt(v + eps)

    x0 = x[0]
    xln = ln(x0) * (1.0 + sa) + sha

    n_local_heads = Wq.shape[1] // 128
    Q = (xln @ Wq).reshape(S, n_local_heads, 128)
    K = (xln @ Wk).reshape(S, n_local_heads, 128)
    V = (xln @ Wv).reshape(S, n_local_heads, 128)

    scores = jnp.einsum("ihd,jhd->hij", Q, K) * 0.08838834764831843
    p = jax.nn.softmax(scores, axis=-1)
    attn = jnp.einsum("hij,jhd->ihd", p, V).reshape(S, n_local_heads * 128)

    attn_out = _ring_allreduce(attn @ Wo)
    x_mid = x0 + ga * attn_out

    xln2 = ln(x_mid) * (1.0 + sm) + shm
    h = xln2 @ W_ff1
    h = h * jax.nn.sigmoid(h)
    ffn = _ring_allreduce(h @ W_ff2)
    out = x_mid + gm * ffn
    return out[None]


# device time: 1064863 ns/iter; 1.3664x vs baseline; 1.1209x over previous
import jax
import jax.numpy as jnp
from jax import lax
from jax.experimental import pallas as pl
from jax.experimental.pallas import tpu as pltpu

N_DEV = 16
S = 4096
D = 1024
HC = S // (2 * N_DEV)
H1 = S // 2


def _ring_allreduce(partial):

    def body(
        x_ref,
        out_ref,
        comm_cw,
        comm_ccw,
        send_cw,
        recv_cw,
        send_ccw,
        recv_ccw,
        credit_cw,
        credit_ccw,
    ):
        my = lax.axis_index("i")
        left = jnp.mod(my - 1, N_DEV)
        right = jnp.mod(my + 1, N_DEV)

        out_ref[...] = x_ref[...]

        n_steps = 2 * (N_DEV - 1)
        for step in range(n_steps):
            slot = step % 2
            is_rs = step < N_DEV - 1
            if is_rs:
                s_cw = jnp.mod(my - step, N_DEV)
                r_cw = jnp.mod(my - step - 1, N_DEV)
                s_ccw = jnp.mod(my + step, N_DEV)
                r_ccw = jnp.mod(my + step + 1, N_DEV)
                dst_cw = comm_cw.at[slot]
                dst_ccw = comm_ccw.at[slot]
            else:
                t = step - (N_DEV - 1)
                s_cw = jnp.mod(my + 1 - t, N_DEV)
                r_cw = jnp.mod(my - t, N_DEV)
                s_ccw = jnp.mod(my - 1 + t, N_DEV)
                r_ccw = jnp.mod(my + t, N_DEV)
                dst_cw = out_ref.at[pl.ds(s_cw * HC, HC), :]
                dst_ccw = out_ref.at[pl.ds(H1 + s_ccw * HC, HC), :]

            if step >= 2:
                pl.semaphore_wait(credit_cw, 1)
                pl.semaphore_wait(credit_ccw, 1)

            rdma_cw = pltpu.make_async_remote_copy(
                src_ref=out_ref.at[pl.ds(s_cw * HC, HC), :],
                dst_ref=dst_cw,
                send_sem=send_cw.at[slot],
                recv_sem=recv_cw.at[slot],
                device_id=(right,),
                device_id_type=pl.DeviceIdType.MESH,
            )
            rdma_ccw = pltpu.make_async_remote_copy(
                src_ref=out_ref.at[pl.ds(H1 + s_ccw * HC, HC), :],
                dst_ref=dst_ccw,
                send_sem=send_ccw.at[slot],
                recv_sem=recv_ccw.at[slot],
                device_id=(left,),
                device_id_type=pl.DeviceIdType.MESH,
            )
            rdma_cw.start()
            rdma_ccw.start()
            rdma_cw.wait()
            rdma_ccw.wait()

            if is_rs:
                out_ref[pl.ds(r_cw * HC, HC), :] += comm_cw[slot]
                out_ref[pl.ds(H1 + r_ccw * HC, HC), :] += comm_ccw[slot]

            pl.semaphore_signal(
                credit_cw,
                inc=1,
                device_id=(left,),
                device_id_type=pl.DeviceIdType.MESH,
            )
            pl.semaphore_signal(
                credit_ccw,
                inc=1,
                device_id=(right,),
                device_id_type=pl.DeviceIdType.MESH,
            )

        pl.semaphore_wait(credit_cw, 2)
        pl.semaphore_wait(credit_ccw, 2)

    return pl.pallas_call(
        body,
        out_shape=jax.ShapeDtypeStruct((S, D), jnp.float32),
        in_specs=[pl.BlockSpec(memory_space=pltpu.VMEM)],
        out_specs=pl.BlockSpec(memory_space=pltpu.VMEM),
        scratch_shapes=[
            pltpu.VMEM((2, HC, D), jnp.float32),
            pltpu.VMEM((2, HC, D), jnp.float32),
            pltpu.SemaphoreType.DMA((2,)),
            pltpu.SemaphoreType.DMA((2,)),
            pltpu.SemaphoreType.DMA((2,)),
            pltpu.SemaphoreType.DMA((2,)),
            pltpu.SemaphoreType.REGULAR,
            pltpu.SemaphoreType.REGULAR,
        ],
    )(partial)


_QB = 512
_DH = 128


def _attention(Q, K, V):
    n_heads = Q.shape[1] // _DH

    def body(q_ref, k_ref, v_ref, o_ref):
        s = lax.dot_general(
            q_ref[...],
            k_ref[...],
            (((1,), (1,)), ((), ())),
            preferred_element_type=jnp.float32,
        ) * 0.08838834764831843
        m = jnp.max(s, axis=-1, keepdims=True)
        p = jnp.exp(s - m)
        l = jnp.sum(p, axis=-1, keepdims=True)
        o = jnp.dot(p, v_ref[...], preferred_element_type=jnp.float32)
        o_ref[...] = o / l

    return pl.pallas_call(
        body,
        grid=(n_heads, S // _QB),
        in_specs=[
            pl.BlockSpec((_QB, _DH), lambda h, qb: (qb, h)),
            pl.BlockSpec((S, _DH), lambda h, qb: (0, h)),
            pl.BlockSpec((S, _DH), lambda h, qb: (0, h)),
        ],
        out_specs=pl.BlockSpec((_QB, _DH), lambda h, qb: (qb, h)),
        out_shape=jax.ShapeDtypeStruct((S, n_heads * _DH), jnp.float32),
    )(Q, K, V)


def kernel(x, Wq, Wk, Wv, Wo, t_emb, W_mod, W_ff1, W_ff2):
    eps = 1e-5
    mod = t_emb @ W_mod
    sa, sha, ga, sm, shm, gm = jnp.split(mod, 6, axis=-1)

    def ln(h):
        m = h.mean(axis=-1, keepdims=True)
        v = h.var(axis=-1, keepdims=True)
        return (h - m) * lax.rsqrt(v + eps)

    x0 = x[0]
    xln = ln(x0) * (1.0 + sa) + sha

    attn = _attention(xln @ Wq, xln @ Wk, xln @ Wv)

    attn_out = _ring_allreduce(attn @ Wo)
    x_mid = x0 + ga * attn_out

    xln2 = ln(x_mid) * (1.0 + sm) + shm
    h = xln2 @ W_ff1
    h = h * jax.nn.sigmoid(h)
    ffn = _ring_allreduce(h @ W_ff2)
    out = x_mid + gm * ffn
    return out[None]


# device time: 868536 ns/iter; 1.6752x vs baseline; 1.2260x over previous
import jax
import jax.numpy as jnp
from jax import lax
from jax.experimental import pallas as pl
from jax.experimental.pallas import tpu as pltpu

N_DEV = 16
S = 4096
D = 1024
HC = S // (2 * N_DEV)
H1 = S // 2


def _mm(a, b):
    return jnp.dot(
        a.astype(jnp.bfloat16),
        b.astype(jnp.bfloat16),
        preferred_element_type=jnp.float32,
    )


def _ring_allreduce(partial):

    def body(
        x_ref,
        out_ref,
        comm_cw,
        comm_ccw,
        send_cw,
        recv_cw,
        send_ccw,
        recv_ccw,
        credit_cw,
        credit_ccw,
    ):
        my = lax.axis_index("i")
        left = jnp.mod(my - 1, N_DEV)
        right = jnp.mod(my + 1, N_DEV)

        out_ref[...] = x_ref[...]

        n_steps = 2 * (N_DEV - 1)
        for step in range(n_steps):
            slot = step % 2
            is_rs = step < N_DEV - 1
            if is_rs:
                s_cw = jnp.mod(my - step, N_DEV)
                r_cw = jnp.mod(my - step - 1, N_DEV)
                s_ccw = jnp.mod(my + step, N_DEV)
                r_ccw = jnp.mod(my + step + 1, N_DEV)
                dst_cw = comm_cw.at[slot]
                dst_ccw = comm_ccw.at[slot]
            else:
                t = step - (N_DEV - 1)
                s_cw = jnp.mod(my + 1 - t, N_DEV)
                r_cw = jnp.mod(my - t, N_DEV)
                s_ccw = jnp.mod(my - 1 + t, N_DEV)
                r_ccw = jnp.mod(my + t, N_DEV)
                dst_cw = out_ref.at[pl.ds(s_cw * HC, HC), :]
                dst_ccw = out_ref.at[pl.ds(H1 + s_ccw * HC, HC), :]

            if step >= 2:
                pl.semaphore_wait(credit_cw, 1)
                pl.semaphore_wait(credit_ccw, 1)

            rdma_cw = pltpu.make_async_remote_copy(
                src_ref=out_ref.at[pl.ds(s_cw * HC, HC), :],
                dst_ref=dst_cw,
                send_sem=send_cw.at[slot],
                recv_sem=recv_cw.at[slot],
                device_id=(right,),
                device_id_type=pl.DeviceIdType.MESH,
            )
            rdma_ccw = pltpu.make_async_remote_copy(
                src_ref=out_ref.at[pl.ds(H1 + s_ccw * HC, HC), :],
                dst_ref=dst_ccw,
                send_sem=send_ccw.at[slot],
                recv_sem=recv_ccw.at[slot],
                device_id=(left,),
                device_id_type=pl.DeviceIdType.MESH,
            )
            rdma_cw.start()
            rdma_ccw.start()
            rdma_cw.wait()
            rdma_ccw.wait()

            if is_rs:
                out_ref[pl.ds(r_cw * HC, HC), :] += comm_cw[slot]
                out_ref[pl.ds(H1 + r_ccw * HC, HC), :] += comm_ccw[slot]

            pl.semaphore_signal(
                credit_cw,
                inc=1,
                device_id=(left,),
                device_id_type=pl.DeviceIdType.MESH,
            )
            pl.semaphore_signal(
                credit_ccw,
                inc=1,
                device_id=(right,),
                device_id_type=pl.DeviceIdType.MESH,
            )

        pl.semaphore_wait(credit_cw, 2)
        pl.semaphore_wait(credit_ccw, 2)

    return pl.pallas_call(
        body,
        out_shape=jax.ShapeDtypeStruct((S, D), jnp.bfloat16),
        in_specs=[pl.BlockSpec(memory_space=pltpu.VMEM)],
        out_specs=pl.BlockSpec(memory_space=pltpu.VMEM),
        scratch_shapes=[
            pltpu.VMEM((2, HC, D), jnp.bfloat16),
            pltpu.VMEM((2, HC, D), jnp.bfloat16),
            pltpu.SemaphoreType.DMA((2,)),
            pltpu.SemaphoreType.DMA((2,)),
            pltpu.SemaphoreType.DMA((2,)),
            pltpu.SemaphoreType.DMA((2,)),
            pltpu.SemaphoreType.REGULAR,
            pltpu.SemaphoreType.REGULAR,
        ],
    )(partial)


_QB = 512
_DH = 128


def _attention(Q, K, V):
    n_heads = Q.shape[1] // _DH

    def body(q_ref, k_ref, v_ref, o_ref):
        s = lax.dot_general(
            q_ref[...],
            k_ref[...],
            (((1,), (1,)), ((), ())),
            preferred_element_type=jnp.float32,
        ) * 0.08838834764831843
        m = jnp.max(s, axis=-1, keepdims=True)
        p = jnp.exp(s - m)
        l = jnp.sum(p, axis=-1, keepdims=True)
        o = jnp.dot(
            p.astype(jnp.bfloat16),
            v_ref[...],
            preferred_element_type=jnp.float32,
        )
        o_ref[...] = (o / l).astype(jnp.bfloat16)

    return pl.pallas_call(
        body,
        grid=(n_heads, S // _QB),
        in_specs=[
            pl.BlockSpec((_QB, _DH), lambda h, qb: (qb, h)),
            pl.BlockSpec((S, _DH), lambda h, qb: (0, h)),
            pl.BlockSpec((S, _DH), lambda h, qb: (0, h)),
        ],
        out_specs=pl.BlockSpec((_QB, _DH), lambda h, qb: (qb, h)),
        out_shape=jax.ShapeDtypeStruct((S, n_heads * _DH), jnp.bfloat16),
    )(Q, K, V)


def kernel(x, Wq, Wk, Wv, Wo, t_emb, W_mod, W_ff1, W_ff2):
    eps = 1e-5
    mod = t_emb @ W_mod
    sa, sha, ga, sm, shm, gm = jnp.split(mod, 6, axis=-1)

    def ln(h):
        m = h.mean(axis=-1, keepdims=True)
        v = h.var(axis=-1, keepdims=True)
        return (h - m) * lax.rsqrt(v + eps)

    x0 = x[0]
    xln = ln(x0) * (1.0 + sa) + sha

    bf = jnp.bfloat16
    xlnb = xln.astype(bf)
    attn = _attention(
        _mm(xlnb, Wq).astype(bf),
        _mm(xlnb, Wk).astype(bf),
        _mm(xlnb, Wv).astype(bf),
    )

    attn_out = _ring_allreduce(_mm(attn, Wo).astype(bf))
    x_mid = x0 + ga * attn_out.astype(jnp.float32)

    xln2 = ln(x_mid) * (1.0 + sm) + shm
    h = _mm(xln2, W_ff1)
    h = h * jax.nn.sigmoid(h)
    ffn = _ring_allreduce(_mm(h, W_ff2).astype(bf))
    out = x_mid + gm * ffn.astype(jnp.float32)
    return out[None]
